# baseline (speedup 1.0000x reference)
import jax
import jax.numpy as jnp
from jax.experimental import pallas as pl
from jax.experimental.pallas import tpu as pltpu
from jax.experimental.pallas import tpu_sc as plsc

_SCALE = 20.0
_LOG2E = 1.4426950408889634
_LN2 = 0.6931471805599453
_BT = 512
_CT = 384
_W = 64
_NSUB = 32


def _lse_kernel(x_ref, lutf_ref, o_ref, c1_ref, c2_ref, lut16_ref,
                sem1, sem2, *, chunks, bt, batch):
    i = pl.program_id(0)
    n = pl.num_programs(0)

    @pl.when(i == 0)
    def _prep():
        pltpu.make_async_copy(lutf_ref, c1_ref, sem1).start()
        pltpu.make_async_copy(lutf_ref, c2_ref, sem2).start()
        for c0, csz in chunks:
            lut16_ref[c0:c0 + csz, :] = (
                lutf_ref[c0:c0 + csz, :] * (_SCALE * _LOG2E)
            ).astype(jnp.bfloat16)

    x = x_ref[...].astype(jnp.bfloat16)
    m = jnp.full((bt, 1), -jnp.inf, jnp.float32)
    s = jnp.zeros((bt, 1), jnp.float32)
    for c0, csz in chunks:
        lut_blk = lut16_ref[c0:c0 + csz, :]
        l2 = jax.lax.dot_general(
            x, lut_blk, (((1,), (1,)), ((), ())),
            preferred_element_type=jnp.float32)
        cmax = jnp.max(l2, axis=1, keepdims=True)
        mn = jnp.maximum(m, cmax)
        e = jnp.exp2(l2 - mn)
        s = s * jnp.exp2(m - mn) + jnp.sum(e, axis=1, keepdims=True)
        m = mn
    part = (jnp.sum(m + jnp.log2(s)) * (_LN2 / batch)).reshape(1, 1)

    @pl.when(i == 0)
    def _init():
        o_ref[...] = jnp.zeros((1, 1), jnp.float32)

    o_ref[...] += part

    @pl.when(i == n - 1)
    def _done():
        pltpu.make_async_copy(lutf_ref, c1_ref, sem1).wait()
        pltpu.make_async_copy(lutf_ref, c2_ref, sem2).wait()


def _sc_tgt_kernel(x_hbm, t_hbm, lut_hbm, o_hbm, idx_vm, g_vm, x_vm,
                   acc_vm, sem_i, sem_g, sem_x, sem_o, *, rows_per_sub,
                   n_feat):
    c = jax.lax.axis_index("c")
    s = jax.lax.axis_index("s")
    sub = c * 16 + s
    acc_vm[...] = jnp.zeros((16,), jnp.float32)
    n_windows = rows_per_sub // _W

    @pl.loop(0, n_windows)
    def _win(w):
        row0 = sub * rows_per_sub + w * _W
        cp_i = pltpu.async_copy(t_hbm.at[pl.ds(row0, _W)], idx_vm, sem_i)
        cp_x = pltpu.async_copy(x_hbm.at[pl.ds(row0, _W)], x_vm, sem_x)
        cp_i.wait()
        cp_g = pltpu.async_copy(lut_hbm.at[idx_vm], g_vm, sem_g)
        cp_x.wait()
        cp_g.wait()

        def row_body(r, _):
            def ch_body(ch, acc):
                xa = x_vm[r, pl.ds(ch * 16, 16)]
                ga = g_vm[r, pl.ds(ch * 16, 16)]
                return acc + xa * ga
            racc = jax.lax.fori_loop(0, n_feat // 16, ch_body,
                                     jnp.zeros((16,), jnp.float32))
            acc_vm[...] = acc_vm[...] + racc
            return 0

        jax.lax.fori_loop(0, _W, row_body, 0)

    pltpu.async_copy(acc_vm, o_hbm.at[sub], sem_o).wait()


def _sc_target_partials(inputs, targets, lut_icc):
    b, f = inputs.shape
    rows_per_sub = b // _NSUB

    @pl.kernel(
        out_type=jax.ShapeDtypeStruct((_NSUB, 16), jnp.float32),
        mesh=plsc.VectorSubcoreMesh(core_axis_name="c",
                                    subcore_axis_name="s"),
        scratch_types=[
            pltpu.VMEM((_W,), jnp.int32),
            pltpu.VMEM((_W, f), jnp.float32),
            pltpu.VMEM((_W, f), jnp.float32),
            pltpu.VMEM((16,), jnp.float32),
            pltpu.SemaphoreType.DMA,
            pltpu.SemaphoreType.DMA,
            pltpu.SemaphoreType.DMA,
            pltpu.SemaphoreType.DMA,
        ],
    )
    def sc_kernel(x_hbm, t_hbm, lut_hbm, o_hbm, idx_vm, g_vm, x_vm,
                  acc_vm, sem_i, sem_g, sem_x, sem_o):
        _sc_tgt_kernel(x_hbm, t_hbm, lut_hbm, o_hbm, idx_vm, g_vm, x_vm,
                       acc_vm, sem_i, sem_g, sem_x, sem_o,
                       rows_per_sub=rows_per_sub, n_feat=f)

    return sc_kernel(inputs, targets, lut_icc)


def kernel(inputs, targets, lut_ccc, lut_icc):
    b, f = inputs.shape
    n_classes = lut_icc.shape[0]
    bt = _BT if b % _BT == 0 else b
    chunks = []
    c0 = 0
    while c0 < n_classes:
        csz = min(_CT, ((n_classes - c0 + 7) // 8) * 8)
        chunks.append((c0, csz))
        c0 += csz
    cp = c0
    assert cp == n_classes, "n_classes must be a multiple of 8"
    lse_sum, lut_out1, lut_out2 = pl.pallas_call(
        lambda xr, lr, orf, c1, c2, l16, s1, s2: _lse_kernel(
            xr, lr, orf, c1, c2, l16, s1, s2, chunks=chunks, bt=bt,
            batch=b),
        grid=(b // bt,),
        in_specs=[
            pl.BlockSpec((bt, f), lambda i: (i, 0)),
            pl.BlockSpec((cp, f), lambda i: (0, 0)),
        ],
        out_specs=[
            pl.BlockSpec((1, 1), lambda i: (0, 0)),
            pl.BlockSpec(memory_space=pltpu.MemorySpace.HBM),
            pl.BlockSpec(memory_space=pltpu.MemorySpace.HBM),
        ],
        out_shape=[
            jax.ShapeDtypeStruct((1, 1), jnp.float32),
            jax.ShapeDtypeStruct((n_classes, f), jnp.float32),
            jax.ShapeDtypeStruct((n_classes, f), jnp.float32),
        ],
        scratch_shapes=[
            pltpu.VMEM((cp, f), jnp.bfloat16),
            pltpu.SemaphoreType.DMA,
            pltpu.SemaphoreType.DMA,
        ],
    )(inputs, lut_icc)
    tgt_partials = _sc_target_partials(inputs, targets, lut_icc)
    loss = lse_sum[0, 0] - (_SCALE / b) * jnp.sum(tgt_partials)
    return (loss, lut_out1, lut_out2)


# speedup vs baseline: 1.0924x; 1.0924x over previous
import jax
import jax.numpy as jnp
from jax.experimental import pallas as pl
from jax.experimental.pallas import tpu as pltpu
from jax.experimental.pallas import tpu_sc as plsc

_SCALE = 20.0
_LOG2E = 1.4426950408889634
_LN2 = 0.6931471805599453
_BT = 512
_CT = 1024
_W = 64
_NSUB = 32


def _lse_kernel(x_ref, lutf_ref, o_ref, c1_ref, c2_ref, lut16_ref,
                sem1, sem2, *, chunks, bt, batch):
    i = pl.program_id(0)
    n = pl.num_programs(0)

    @pl.when(i == 0)
    def _prep():
        pltpu.make_async_copy(lutf_ref, c1_ref, sem1).start()
        pltpu.make_async_copy(lutf_ref, c2_ref, sem2).start()
        for c0, csz in chunks:
            lut16_ref[c0:c0 + csz, :] = (
                lutf_ref[c0:c0 + csz, :] * (_SCALE * _LOG2E)
            ).astype(jnp.bfloat16)

    x = x_ref[...].astype(jnp.bfloat16)
    m = jnp.full((bt, 1), -jnp.inf, jnp.float32)
    s = jnp.zeros((bt, 1), jnp.float32)
    for c0, csz in chunks:
        lut_blk = lut16_ref[c0:c0 + csz, :]
        l2 = jax.lax.dot_general(
            x, lut_blk, (((1,), (1,)), ((), ())),
            preferred_element_type=jnp.float32)
        cmax = jnp.max(l2, axis=1, keepdims=True)
        mn = jnp.maximum(m, cmax)
        e = jnp.exp2(l2 - mn)
        s = s * jnp.exp2(m - mn) + jnp.sum(e, axis=1, keepdims=True)
        m = mn
    part = (jnp.sum(m + jnp.log2(s)) * (_LN2 / batch)).reshape(1, 1)

    @pl.when(i == 0)
    def _init():
        o_ref[...] = jnp.zeros((1, 1), jnp.float32)

    o_ref[...] += part

    @pl.when(i == n - 1)
    def _done():
        pltpu.make_async_copy(lutf_ref, c1_ref, sem1).wait()
        pltpu.make_async_copy(lutf_ref, c2_ref, sem2).wait()


def _sc_tgt_kernel(x_hbm, t_hbm, lut_hbm, o_hbm, idx_vm, g_vm, x_vm,
                   acc_vm, sem_i, sem_g, sem_x, sem_o, *, rows_per_sub,
                   n_feat):
    c = jax.lax.axis_index("c")
    s = jax.lax.axis_index("s")
    sub = c * 16 + s
    acc_vm[...] = jnp.zeros((16,), jnp.float32)
    n_windows = rows_per_sub // _W

    @pl.loop(0, n_windows)
    def _win(w):
        row0 = sub * rows_per_sub + w * _W
        cp_i = pltpu.async_copy(t_hbm.at[pl.ds(row0, _W)], idx_vm, sem_i)
        cp_x = pltpu.async_copy(x_hbm.at[pl.ds(row0, _W)], x_vm, sem_x)
        cp_i.wait()
        cp_g = pltpu.async_copy(lut_hbm.at[idx_vm], g_vm, sem_g)
        cp_x.wait()
        cp_g.wait()

        def row_body(r, _):
            def ch_body(ch, acc):
                xa = x_vm[r, pl.ds(ch * 16, 16)]
                ga = g_vm[r, pl.ds(ch * 16, 16)]
                return acc + xa * ga
            racc = jax.lax.fori_loop(0, n_feat // 16, ch_body,
                                     jnp.zeros((16,), jnp.float32))
            acc_vm[...] = acc_vm[...] + racc
            return 0

        jax.lax.fori_loop(0, _W, row_body, 0)

    pltpu.async_copy(acc_vm, o_hbm.at[sub], sem_o).wait()


def _sc_target_partials(inputs, targets, lut_icc):
    b, f = inputs.shape
    rows_per_sub = b // _NSUB

    @pl.kernel(
        out_type=jax.ShapeDtypeStruct((_NSUB, 16), jnp.float32),
        mesh=plsc.VectorSubcoreMesh(core_axis_name="c",
                                    subcore_axis_name="s"),
        scratch_types=[
            pltpu.VMEM((_W,), jnp.int32),
            pltpu.VMEM((_W, f), jnp.float32),
            pltpu.VMEM((_W, f), jnp.float32),
            pltpu.VMEM((16,), jnp.float32),
            pltpu.SemaphoreType.DMA,
            pltpu.SemaphoreType.DMA,
            pltpu.SemaphoreType.DMA,
            pltpu.SemaphoreType.DMA,
        ],
    )
    def sc_kernel(x_hbm, t_hbm, lut_hbm, o_hbm, idx_vm, g_vm, x_vm,
                  acc_vm, sem_i, sem_g, sem_x, sem_o):
        _sc_tgt_kernel(x_hbm, t_hbm, lut_hbm, o_hbm, idx_vm, g_vm, x_vm,
                       acc_vm, sem_i, sem_g, sem_x, sem_o,
                       rows_per_sub=rows_per_sub, n_feat=f)

    return sc_kernel(inputs, targets, lut_icc)


def kernel(inputs, targets, lut_ccc, lut_icc):
    b, f = inputs.shape
    n_classes = lut_icc.shape[0]
    bt = _BT if b % _BT == 0 else b
    chunks = []
    c0 = 0
    while c0 < n_classes:
        csz = min(_CT, ((n_classes - c0 + 7) // 8) * 8)
        chunks.append((c0, csz))
        c0 += csz
    cp = c0
    assert cp == n_classes, "n_classes must be a multiple of 8"
    lse_sum, lut_out1, lut_out2 = pl.pallas_call(
        lambda xr, lr, orf, c1, c2, l16, s1, s2: _lse_kernel(
            xr, lr, orf, c1, c2, l16, s1, s2, chunks=chunks, bt=bt,
            batch=b),
        grid=(b // bt,),
        in_specs=[
            pl.BlockSpec((bt, f), lambda i: (i, 0)),
            pl.BlockSpec((cp, f), lambda i: (0, 0)),
        ],
        out_specs=[
            pl.BlockSpec((1, 1), lambda i: (0, 0)),
            pl.BlockSpec(memory_space=pltpu.MemorySpace.HBM),
            pl.BlockSpec(memory_space=pltpu.MemorySpace.HBM),
        ],
        out_shape=[
            jax.ShapeDtypeStruct((1, 1), jnp.float32),
            jax.ShapeDtypeStruct((n_classes, f), jnp.float32),
            jax.ShapeDtypeStruct((n_classes, f), jnp.float32),
        ],
        scratch_shapes=[
            pltpu.VMEM((cp, f), jnp.bfloat16),
            pltpu.SemaphoreType.DMA,
            pltpu.SemaphoreType.DMA,
        ],
    )(inputs, lut_icc)
    tgt_partials = _sc_target_partials(inputs, targets, lut_icc)
    loss = lse_sum[0, 0] - (_SCALE / b) * jnp.sum(tgt_partials)
    return (loss, lut_out1, lut_out2)


# speedup vs baseline: 1.1367x; 1.0406x over previous
import jax
import jax.numpy as jnp
from jax.experimental import pallas as pl
from jax.experimental.pallas import tpu as pltpu
from jax.experimental.pallas import tpu_sc as plsc

_SCALE = 20.0
_LOG2E = 1.4426950408889634
_LN2 = 0.6931471805599453
_BT = 256
_CT = 512
_W = 64
_NSUB = 32


def _lse_kernel(x_ref, lutf_ref, o_ref, c1_ref, c2_ref, lut16_ref,
                sem1, sem2, *, chunks, bt, batch):
    i = pl.program_id(0)
    n = pl.num_programs(0)

    @pl.when(i == 0)
    def _prep():
        pltpu.make_async_copy(lutf_ref, c1_ref, sem1).start()
        pltpu.make_async_copy(lutf_ref, c2_ref, sem2).start()
        for c0, csz in chunks:
            lut16_ref[c0:c0 + csz, :] = (
                lutf_ref[c0:c0 + csz, :] * (_SCALE * _LOG2E)
            ).astype(jnp.bfloat16)

    x = x_ref[...].astype(jnp.bfloat16)
    m = jnp.full((bt, 1), -jnp.inf, jnp.float32)
    s = jnp.zeros((bt, 1), jnp.float32)
    for c0, csz in chunks:
        lut_blk = lut16_ref[c0:c0 + csz, :]
        l2 = jax.lax.dot_general(
            x, lut_blk, (((1,), (1,)), ((), ())),
            preferred_element_type=jnp.float32)
        cmax = jnp.max(l2, axis=1, keepdims=True)
        mn = jnp.maximum(m, cmax)
        e = jnp.exp2(l2 - mn)
        s = s * jnp.exp2(m - mn) + jnp.sum(e, axis=1, keepdims=True)
        m = mn
    part = (jnp.sum(m + jnp.log2(s)) * (_LN2 / batch)).reshape(1, 1)

    @pl.when(i == 0)
    def _init():
        o_ref[...] = jnp.zeros((1, 1), jnp.float32)

    o_ref[...] += part

    @pl.when(i == n - 1)
    def _done():
        pltpu.make_async_copy(lutf_ref, c1_ref, sem1).wait()
        pltpu.make_async_copy(lutf_ref, c2_ref, sem2).wait()


def _sc_tgt_kernel(x_hbm, t_hbm, lut_hbm, o_hbm, idx_vm, g_vm, x_vm,
                   acc_vm, sem_i, sem_g, sem_x, sem_o, *, rows_per_sub,
                   n_feat):
    c = jax.lax.axis_index("c")
    s = jax.lax.axis_index("s")
    sub = c * 16 + s
    acc_vm[...] = jnp.zeros((16,), jnp.float32)
    n_windows = rows_per_sub // _W

    @pl.loop(0, n_windows)
    def _win(w):
        row0 = sub * rows_per_sub + w * _W
        cp_i = pltpu.async_copy(t_hbm.at[pl.ds(row0, _W)], idx_vm, sem_i)
        cp_x = pltpu.async_copy(x_hbm.at[pl.ds(row0, _W)], x_vm, sem_x)
        cp_i.wait()
        cp_g = pltpu.async_copy(lut_hbm.at[idx_vm], g_vm, sem_g)
        cp_x.wait()
        cp_g.wait()

        def row_body(r, _):
            def ch_body(ch, acc):
                xa = x_vm[r, pl.ds(ch * 16, 16)]
                ga = g_vm[r, pl.ds(ch * 16, 16)]
                return acc + xa * ga
            racc = jax.lax.fori_loop(0, n_feat // 16, ch_body,
                                     jnp.zeros((16,), jnp.float32))
            acc_vm[...] = acc_vm[...] + racc
            return 0

        jax.lax.fori_loop(0, _W, row_body, 0)

    pltpu.async_copy(acc_vm, o_hbm.at[sub], sem_o).wait()


def _sc_target_partials(inputs, targets, lut_icc):
    b, f = inputs.shape
    rows_per_sub = b // _NSUB

    @pl.kernel(
        out_type=jax.ShapeDtypeStruct((_NSUB, 16), jnp.float32),
        mesh=plsc.VectorSubcoreMesh(core_axis_name="c",
                                    subcore_axis_name="s"),
        scratch_types=[
            pltpu.VMEM((_W,), jnp.int32),
            pltpu.VMEM((_W, f), jnp.float32),
            pltpu.VMEM((_W, f), jnp.float32),
            pltpu.VMEM((16,), jnp.float32),
            pltpu.SemaphoreType.DMA,
            pltpu.SemaphoreType.DMA,
            pltpu.SemaphoreType.DMA,
            pltpu.SemaphoreType.DMA,
        ],
    )
    def sc_kernel(x_hbm, t_hbm, lut_hbm, o_hbm, idx_vm, g_vm, x_vm,
                  acc_vm, sem_i, sem_g, sem_x, sem_o):
        _sc_tgt_kernel(x_hbm, t_hbm, lut_hbm, o_hbm, idx_vm, g_vm, x_vm,
                       acc_vm, sem_i, sem_g, sem_x, sem_o,
                       rows_per_sub=rows_per_sub, n_feat=f)

    return sc_kernel(inputs, targets, lut_icc)


def kernel(inputs, targets, lut_ccc, lut_icc):
    b, f = inputs.shape
    n_classes = lut_icc.shape[0]
    bt = _BT if b % _BT == 0 else b
    chunks = []
    c0 = 0
    while c0 < n_classes:
        csz = min(_CT, ((n_classes - c0 + 7) // 8) * 8)
        chunks.append((c0, csz))
        c0 += csz
    cp = c0
    assert cp == n_classes, "n_classes must be a multiple of 8"
    lse_sum, lut_out1, lut_out2 = pl.pallas_call(
        lambda xr, lr, orf, c1, c2, l16, s1, s2: _lse_kernel(
            xr, lr, orf, c1, c2, l16, s1, s2, chunks=chunks, bt=bt,
            batch=b),
        grid=(b // bt,),
        in_specs=[
            pl.BlockSpec((bt, f), lambda i: (i, 0)),
            pl.BlockSpec((cp, f), lambda i: (0, 0)),
        ],
        out_specs=[
            pl.BlockSpec((1, 1), lambda i: (0, 0)),
            pl.BlockSpec(memory_space=pltpu.MemorySpace.HBM),
            pl.BlockSpec(memory_space=pltpu.MemorySpace.HBM),
        ],
        out_shape=[
            jax.ShapeDtypeStruct((1, 1), jnp.float32),
            jax.ShapeDtypeStruct((n_classes, f), jnp.float32),
            jax.ShapeDtypeStruct((n_classes, f), jnp.float32),
        ],
        scratch_shapes=[
            pltpu.VMEM((cp, f), jnp.bfloat16),
            pltpu.SemaphoreType.DMA,
            pltpu.SemaphoreType.DMA,
        ],
    )(inputs, lut_icc)
    tgt_partials = _sc_target_partials(inputs, targets, lut_icc)
    loss = lse_sum[0, 0] - (_SCALE / b) * jnp.sum(tgt_partials)
    return (loss, lut_out1, lut_out2)


# speedup vs baseline: 1.1951x; 1.0513x over previous
import jax
import jax.numpy as jnp
from jax.experimental import pallas as pl
from jax.experimental.pallas import tpu as pltpu
from jax.experimental.pallas import tpu_sc as plsc

_SCALE = 20.0
_LOG2E = 1.4426950408889634
_LN2 = 0.6931471805599453
_BT = 1024
_CT = 512
_W = 64
_NSUB = 32


def _lse_kernel(x_ref, lutf_ref, o_ref, c1_ref, c2_ref, lut16_ref,
                sem1, sem2, *, chunks, bt, batch):
    i = pl.program_id(0)
    n = pl.num_programs(0)

    @pl.when(i == 0)
    def _prep():
        pltpu.make_async_copy(lutf_ref, c1_ref, sem1).start()
        pltpu.make_async_copy(lutf_ref, c2_ref, sem2).start()
        for c0, csz in chunks:
            lut16_ref[c0:c0 + csz, :] = (
                lutf_ref[c0:c0 + csz, :] * (_SCALE * _LOG2E)
            ).astype(jnp.bfloat16)

    x = x_ref[...].astype(jnp.bfloat16)
    m = jnp.full((bt, 1), -jnp.inf, jnp.float32)
    s = jnp.zeros((bt, 1), jnp.float32)
    for c0, csz in chunks:
        lut_blk = lut16_ref[c0:c0 + csz, :]
        l2 = jax.lax.dot_general(
            x, lut_blk, (((1,), (1,)), ((), ())),
            preferred_element_type=jnp.float32)
        cmax = jnp.max(l2, axis=1, keepdims=True)
        mn = jnp.maximum(m, cmax)
        e = jnp.exp2(l2 - mn)
        s = s * jnp.exp2(m - mn) + jnp.sum(e, axis=1, keepdims=True)
        m = mn
    part = (jnp.sum(m + jnp.log2(s)) * (_LN2 / batch)).reshape(1, 1)

    @pl.when(i == 0)
    def _init():
        o_ref[...] = jnp.zeros((1, 1), jnp.float32)

    o_ref[...] += part

    @pl.when(i == n - 1)
    def _done():
        pltpu.make_async_copy(lutf_ref, c1_ref, sem1).wait()
        pltpu.make_async_copy(lutf_ref, c2_ref, sem2).wait()


def _sc_tgt_kernel(x_hbm, t_hbm, lut_hbm, o_hbm, idx_vm, g_vm, x_vm,
                   acc_vm, sem_i, sem_g, sem_x, sem_o, *, rows_per_sub,
                   n_feat):
    c = jax.lax.axis_index("c")
    s = jax.lax.axis_index("s")
    sub = c * 16 + s
    acc_vm[...] = jnp.zeros((16,), jnp.float32)
    n_windows = rows_per_sub // _W

    @pl.loop(0, n_windows)
    def _win(w):
        row0 = sub * rows_per_sub + w * _W
        cp_i = pltpu.async_copy(t_hbm.at[pl.ds(row0, _W)], idx_vm, sem_i)
        cp_x = pltpu.async_copy(x_hbm.at[pl.ds(row0, _W)], x_vm, sem_x)
        cp_i.wait()
        cp_g = pltpu.async_copy(lut_hbm.at[idx_vm], g_vm, sem_g)
        cp_x.wait()
        cp_g.wait()

        def row_body(r, _):
            def ch_body(ch, acc):
                xa = x_vm[r, pl.ds(ch * 16, 16)]
                ga = g_vm[r, pl.ds(ch * 16, 16)]
                return acc + xa * ga
            racc = jax.lax.fori_loop(0, n_feat // 16, ch_body,
                                     jnp.zeros((16,), jnp.float32))
            acc_vm[...] = acc_vm[...] + racc
            return 0

        jax.lax.fori_loop(0, _W, row_body, 0)

    pltpu.async_copy(acc_vm, o_hbm.at[sub], sem_o).wait()


def _sc_target_partials(inputs, targets, lut_icc):
    b, f = inputs.shape
    rows_per_sub = b // _NSUB

    @pl.kernel(
        out_type=jax.ShapeDtypeStruct((_NSUB, 16), jnp.float32),
        mesh=plsc.VectorSubcoreMesh(core_axis_name="c",
                                    subcore_axis_name="s"),
        scratch_types=[
            pltpu.VMEM((_W,), jnp.int32),
            pltpu.VMEM((_W, f), jnp.float32),
            pltpu.VMEM((_W, f), jnp.float32),
            pltpu.VMEM((16,), jnp.float32),
            pltpu.SemaphoreType.DMA,
            pltpu.SemaphoreType.DMA,
            pltpu.SemaphoreType.DMA,
            pltpu.SemaphoreType.DMA,
        ],
    )
    def sc_kernel(x_hbm, t_hbm, lut_hbm, o_hbm, idx_vm, g_vm, x_vm,
                  acc_vm, sem_i, sem_g, sem_x, sem_o):
        _sc_tgt_kernel(x_hbm, t_hbm, lut_hbm, o_hbm, idx_vm, g_vm, x_vm,
                       acc_vm, sem_i, sem_g, sem_x, sem_o,
                       rows_per_sub=rows_per_sub, n_feat=f)

    return sc_kernel(inputs, targets, lut_icc)


def kernel(inputs, targets, lut_ccc, lut_icc):
    b, f = inputs.shape
    n_classes = lut_icc.shape[0]
    bt = _BT if b % _BT == 0 else b
    chunks = []
    c0 = 0
    while c0 < n_classes:
        csz = min(_CT, ((n_classes - c0 + 7) // 8) * 8)
        chunks.append((c0, csz))
        c0 += csz
    cp = c0
    assert cp == n_classes, "n_classes must be a multiple of 8"
    lse_sum, lut_out1, lut_out2 = pl.pallas_call(
        lambda xr, lr, orf, c1, c2, l16, s1, s2: _lse_kernel(
            xr, lr, orf, c1, c2, l16, s1, s2, chunks=chunks, bt=bt,
            batch=b),
        grid=(b // bt,),
        in_specs=[
            pl.BlockSpec((bt, f), lambda i: (i, 0)),
            pl.BlockSpec((cp, f), lambda i: (0, 0)),
        ],
        out_specs=[
            pl.BlockSpec((1, 1), lambda i: (0, 0)),
            pl.BlockSpec(memory_space=pltpu.MemorySpace.HBM),
            pl.BlockSpec(memory_space=pltpu.MemorySpace.HBM),
        ],
        out_shape=[
            jax.ShapeDtypeStruct((1, 1), jnp.float32),
            jax.ShapeDtypeStruct((n_classes, f), jnp.float32),
            jax.ShapeDtypeStruct((n_classes, f), jnp.float32),
        ],
        scratch_shapes=[
            pltpu.VMEM((cp, f), jnp.bfloat16),
            pltpu.SemaphoreType.DMA,
            pltpu.SemaphoreType.DMA,
        ],
    )(inputs, lut_icc)
    tgt_partials = _sc_target_partials(inputs, targets, lut_icc)
    loss = lse_sum[0, 0] - (_SCALE / b) * jnp.sum(tgt_partials)
    return (loss, lut_out1, lut_out2)
